# Initial kernel scaffold; baseline (speedup 1.0000x reference)
import jax
import jax.numpy as jnp
from jax.experimental import pallas as pl


def kernel(concept_ids, relation, head, tail, triple_label, emb_table, W_s0, W_s1, W_n0, W_n1, W_r0, W_r1, W_triple, L_lin, Wih_f, Whh_f, bih_f, bhh_f, Wih_b, Whh_b, bih_b, bhh_b, L_cause):
    raise NotImplementedError("write your pallas kernel here")


# speedup vs baseline: 22.4060x; 22.4060x over previous
import functools

import jax
import jax.numpy as jnp
from jax import lax
from jax.experimental import pallas as pl
from jax.experimental.pallas import tpu as pltpu
from jax.experimental.pallas import tpu_sc as plsc

B, G, M, T, D, H, V = 8, 4, 512, 1024, 256, 256, 50000
BG = B * G
N_IDX = BG * M + BG * T


def _sc_gather(table, idx):
    n = idx.shape[0]
    d = table.shape[1]
    info = plsc.get_sparse_core_info()
    nc, ns = info.num_cores, info.num_subcores
    nw = nc * ns
    per_w = n // nw
    chunk = 128
    n_ch = per_w // chunk
    mesh = plsc.VectorSubcoreMesh(core_axis_name="c", subcore_axis_name="s")

    @functools.partial(
        pl.kernel,
        mesh=mesh,
        out_type=jax.ShapeDtypeStruct((n, d), jnp.float32),
        scratch_types=[
            pltpu.VMEM((chunk,), jnp.int32),
            pltpu.VMEM((chunk, d), jnp.float32),
            pltpu.SemaphoreType.DMA,
        ],
    )
    def k(table_hbm, idx_hbm, out_hbm, idx_v, rows_v, sem):
        wid = lax.axis_index("s") * nc + lax.axis_index("c")
        base = wid * per_w

        def body(i, carry):
            off = base + i * chunk
            pltpu.sync_copy(idx_hbm.at[pl.ds(off, chunk)], idx_v)
            pltpu.async_copy(table_hbm.at[idx_v], rows_v, sem).wait()
            pltpu.sync_copy(rows_v, out_hbm.at[pl.ds(off, chunk)])
            return carry

        lax.fori_loop(0, n_ch, body, 0)

    return k(table, idx)


def _gcn_body(ch_ref, rh_ref, hd_ref, tl_ref, lbl_ref,
              ws0_ref, wn0_ref, ws1_ref, wn1_ref, wr0_ref, wr1_ref,
              wt_h_ref, wt_r_ref, wt_t_ref, llin_ref,
              triple_ref, cause_ref):
    f32 = jnp.float32
    hd = hd_ref[0, 0, :]
    tl = tl_ref[0, 0, :]
    lbl = lbl_ref[0, 0, :]
    cnt = (lbl != -1).astype(f32)

    iota_tm = lax.broadcasted_iota(jnp.int32, (T, M), 1)
    iota_mt = lax.broadcasted_iota(jnp.int32, (M, T), 0)
    sh_p = (iota_tm == hd[:, None]).astype(f32)
    st_p = (iota_tm == tl[:, None]).astype(f32)
    shm_t = (iota_mt == hd[None, :]).astype(f32) * cnt[None, :]
    stm_t = (iota_mt == tl[None, :]).astype(f32) * cnt[None, :]

    dot = functools.partial(jnp.dot, preferred_element_type=f32)

    a = dot(stm_t, sh_p) + dot(shm_t, st_p)
    cnt_out = jnp.sum(a, axis=1)
    c = jnp.maximum(cnt_out, 1.0)[:, None]
    rh0 = rh_ref[0]
    r0 = dot(stm_t + shm_t, rh0)

    ch = ch_ref[0]
    upd = dot(a, ch) - r0
    ch = jax.nn.relu(dot(ch, ws0_ref[...]) + dot(upd, wn0_ref[...]) / c)
    r1 = dot(r0, wr0_ref[...])
    upd = dot(a, ch) - r1
    ch = jax.nn.relu(dot(ch, ws1_ref[...]) + dot(upd, wn1_ref[...]) / c)

    rh2 = dot(dot(rh0, wr0_ref[...]), wr1_ref[...])
    head_r = dot(sh_p, ch)
    tail_r = dot(st_p, ch)
    triple = (dot(head_r, wt_h_ref[...]) + dot(rh2, wt_r_ref[...])
              + dot(tail_r, wt_t_ref[...]))
    triple_ref[0] = triple
    cause_ref[0] = dot(jnp.sum(triple, axis=0, keepdims=True), llin_ref[...])


def _gcn_call(ch0, rh0, hd3, tl3, lbl3, ws0, wn0, ws1, wn1, wr0, wr1,
              wt_h, wt_r, wt_t, llin):
    full = lambda shp: pl.BlockSpec(shp, lambda b: (0,) * len(shp))
    batch3 = lambda shp: pl.BlockSpec((1,) + shp, lambda b: (b, 0, 0))
    return pl.pallas_call(
        _gcn_body,
        grid=(BG,),
        in_specs=[
            batch3((M, D)), batch3((T, D)),
            batch3((1, T)), batch3((1, T)), batch3((1, T)),
            full((D, D)), full((D, D)), full((D, D)), full((D, D)),
            full((D, D)), full((D, D)),
            full((D, D)), full((D, D)), full((D, D)), full((D, D)),
        ],
        out_specs=[batch3((T, D)), batch3((1, D))],
        out_shape=[
            jax.ShapeDtypeStruct((BG, T, D), jnp.float32),
            jax.ShapeDtypeStruct((BG, 1, D), jnp.float32),
        ],
    )(ch0, rh0, hd3, tl3, lbl3, ws0, wn0, ws1, wn1, wr0, wr1,
      wt_h, wt_r, wt_t, llin)


def _gru_body(xs_ref, wih_f_ref, whh_f_ref, bih_f_ref, bhh_f_ref,
              wih_b_ref, whh_b_ref, bih_b_ref, bhh_b_ref,
              lc_b_ref, lc_f_ref, out_ref):
    dot = functools.partial(jnp.dot, preferred_element_type=jnp.float32)

    def run(step_ids, wih, whh, bih, bhh):
        h = jnp.zeros((B, H), jnp.float32)
        for g in step_ids:
            xt = xs_ref[g]
            gx = dot(xt, wih[...]) + bih[...]
            gh = dot(h, whh[...]) + bhh[...]
            r = jax.nn.sigmoid(gx[:, 0:H] + gh[:, 0:H])
            z = jax.nn.sigmoid(gx[:, H:2 * H] + gh[:, H:2 * H])
            n = jnp.tanh(gx[:, 2 * H:] + r * gh[:, 2 * H:])
            h = (1.0 - z) * n + z * h
        return h

    h_f = run(range(G), wih_f_ref, whh_f_ref, bih_f_ref, bhh_f_ref)
    h_b = run(range(G - 1, -1, -1), wih_b_ref, whh_b_ref, bih_b_ref, bhh_b_ref)
    out_ref[...] = jnp.tanh(dot(h_b, lc_b_ref[...]) + dot(h_f, lc_f_ref[...]))


def _gru_call(xs, wih_f, whh_f, bih_f, bhh_f, wih_b, whh_b, bih_b, bhh_b,
              lc_b, lc_f):
    return pl.pallas_call(
        _gru_body,
        out_shape=jax.ShapeDtypeStruct((B, H), jnp.float32),
    )(xs, wih_f, whh_f, bih_f, bhh_f, wih_b, whh_b, bih_b, bhh_b, lc_b, lc_f)


def kernel(concept_ids, relation, head, tail, triple_label, emb_table,
           W_s0, W_s1, W_n0, W_n1, W_r0, W_r1, W_triple, L_lin,
           Wih_f, Whh_f, bih_f, bhh_f, Wih_b, Whh_b, bih_b, bhh_b, L_cause):
    cid = concept_ids.reshape(BG * M)
    rel = relation.reshape(BG * T)
    idx = jnp.concatenate([cid, rel]).astype(jnp.int32)

    rows = _sc_gather(emb_table, idx)
    ch0 = rows[:BG * M].reshape(BG, M, D)
    rh0 = rows[BG * M:].reshape(BG, T, D)

    hd3 = head.reshape(BG, 1, T).astype(jnp.int32)
    tl3 = tail.reshape(BG, 1, T).astype(jnp.int32)
    lbl3 = triple_label.reshape(BG, 1, T).astype(jnp.int32)

    wt_h = W_triple[0:D]
    wt_r = W_triple[D:2 * D]
    wt_t = W_triple[2 * D:]

    triple, cause = _gcn_call(ch0, rh0, hd3, tl3, lbl3,
                              W_s0, W_n0, W_s1, W_n1, W_r0, W_r1,
                              wt_h, wt_r, wt_t, L_lin)

    xs = cause.reshape(B, G, D).transpose(1, 0, 2)

    encoded = _gru_call(xs, Wih_f.T, Whh_f.T, bih_f.reshape(1, 3 * H),
                        bhh_f.reshape(1, 3 * H), Wih_b.T, Whh_b.T,
                        bih_b.reshape(1, 3 * H), bhh_b.reshape(1, 3 * H),
                        L_cause[0:H], L_cause[H:])

    return (triple, encoded)
